# baseline (speedup 1.0000x reference)
import functools

import jax
import jax.numpy as jnp
from jax.experimental import pallas as pl
from jax.experimental.pallas import tpu as pltpu

LANE = 128
_GY = -9.81
_PH = 8


def _fused_body(pos_ref, acc_ref, out_ref):
    f32 = jnp.float32
    X = pos_ref[...]
    A = acc_ref[...]
    nr = X.shape[0]

    Y = X.T.reshape(_PH, 32, nr)
    Z = A.T.reshape(_PH, 32, nr)

    def p(d):
        return Y[:, d, :]

    ea, eb, ec = p(0), p(1), p(2)
    sx, cx = jnp.sin(ea), jnp.cos(ea)
    sy, cy = jnp.sin(eb), jnp.cos(eb)
    sz, cz = jnp.sin(ec), jnp.cos(ec)
    r00 = cz * cy
    r01 = cz * sy * sx - sz * cx
    r02 = cz * sy * cx + sz * sx
    r10 = sz * cy
    r11 = sz * sy * sx + cz * cx
    r12 = sz * sy * cx - cz * sx
    r20 = -sy
    r21 = cy * sx
    r22 = cy * cx

    px, py, pz = p(3), p(4), p(5)

    cax = Z[:, 0, :]
    cay = Z[:, 1, :] - f32(_GY)
    caz = Z[:, 2, :]

    contact = [(p(6 + i) < f32(0.1)).astype(f32) for i in range(2)]
    s = contact[0] + contact[1]
    active = (s > f32(0.0)).astype(f32)
    inv_s = jnp.where(s > f32(0.0), f32(1.0) / jnp.maximum(s, f32(1.0)), f32(0.0))
    fax, fay, faz = cax * inv_s, cay * inv_s, caz * inv_s

    planes = []
    f_planes, c_planes = [], []
    for i in range(2):
        ci = contact[i]
        fx, fy, fz = ci * fax, ci * fay, ci * faz

        f_planes += [r00 * fx + r10 * fy + r20 * fz,
                     r01 * fx + r11 * fy + r21 * fz,
                     r02 * fx + r12 * fy + r22 * fz]

        wcx, wcy, wcz = p(8 + 3 * i), p(9 + 3 * i), p(10 + 3 * i)
        dx, dy, dz = wcx - px, wcy - py, wcz - pz
        c_planes += [active * (r00 * dx + r10 * dy + r20 * dz),
                     active * (r01 * dx + r11 * dy + r21 * dz),
                     active * (r02 * dx + r12 * dy + r22 * dz)]

        mx = wcy * fz - wcz * fy
        my = wcz * fx - wcx * fz
        mz = wcx * fy - wcy * fx

        bfx = r00 * fx + r01 * fy + r02 * fz
        bfy = r10 * fx + r11 * fy + r12 * fz
        bfz = r20 * fx + r21 * fy + r22 * fz
        planes += [r00 * mx + r01 * my + r02 * mz + (py * bfz - pz * bfy),
                   r10 * mx + r11 * my + r12 * mz + (pz * bfx - px * bfz),
                   r20 * mx + r21 * my + r22 * mz + (px * bfy - py * bfx),
                   bfx, bfy, bfz]
    planes += f_planes + c_planes

    T24 = jnp.stack(planes)
    zpad = jnp.zeros((_PH, nr), f32)
    pieces = []
    for j in range(_PH):
        pieces.append(T24[:, j, :])
        pieces.append(zpad)
    out_ref[...] = jnp.concatenate(pieces, axis=0).T


@jax.jit
def _contact_call(pos, acc):
    B, T, D = pos.shape
    TB = 1024
    T_pad = -(-T // TB) * TB
    if T_pad != T:
        padw = ((0, 0), (0, T_pad - T), (0, 0))
        pos = jnp.pad(pos, padw)
        acc = jnp.pad(acc, padw)
    n_t = T_pad // TB
    nr = TB // _PH

    pos_v = pos.reshape(B, T_pad // _PH, _PH * D)
    acc_v = acc.reshape(B, T_pad // _PH, _PH * D)

    idx = lambda b, t: (b, t, 0)
    packed = pl.pallas_call(
        _fused_body,
        grid=(B, n_t),
        in_specs=[pl.BlockSpec((None, nr, _PH * D), idx),
                  pl.BlockSpec((None, nr, _PH * D), idx)],
        out_specs=pl.BlockSpec((None, nr, _PH * D), idx),
        out_shape=jax.ShapeDtypeStruct((B, T_pad // _PH, _PH * D), jnp.float32),
        compiler_params=pltpu.CompilerParams(
            dimension_semantics=("parallel", "parallel")),
    )(pos_v, acc_v)

    V = packed.reshape(B, T_pad, D)
    if T_pad != T:
        V = V[:, :T]
    return V[..., 0:12], V[..., 12:18], V[..., 18:24]


def kernel(pos, vel, acc):
    del vel
    B, T, D = pos.shape
    wrench, force, cop = _contact_call(pos.astype(jnp.float32),
                                       acc.astype(jnp.float32))
    zeros = lambda f: jnp.zeros((B, T, f), jnp.float32)
    return {
        "groundContactWrenchesInRootFrame": wrench,
        "groundContactForcesInRootFrame": force,
        "groundContactCenterOfPressureInRootFrame": cop,
        "groundContactTorquesInRootFrame": zeros(6),
        "residualWrenchInRootFrame": zeros(6),
        "contact": zeros(2),
        "comAccInRootFrame": zeros(3),
        "tau": zeros(D),
    }


# speedup vs baseline: 1.0042x; 1.0042x over previous
import functools

import jax
import jax.numpy as jnp
from jax.experimental import pallas as pl
from jax.experimental.pallas import tpu as pltpu

LANE = 128
_GY = -9.81
_PH = 4


def _fused_body(pos_ref, acc_ref, out_ref):
    f32 = jnp.float32
    X = pos_ref[...]
    A = acc_ref[...]
    nr = X.shape[0]

    Y = X.T.reshape(_PH, 32, nr)
    Z = A.T.reshape(_PH, 32, nr)

    def p(d):
        return Y[:, d, :]

    ea, eb, ec = p(0), p(1), p(2)
    sx, cx = jnp.sin(ea), jnp.cos(ea)
    sy, cy = jnp.sin(eb), jnp.cos(eb)
    sz, cz = jnp.sin(ec), jnp.cos(ec)
    r00 = cz * cy
    r01 = cz * sy * sx - sz * cx
    r02 = cz * sy * cx + sz * sx
    r10 = sz * cy
    r11 = sz * sy * sx + cz * cx
    r12 = sz * sy * cx - cz * sx
    r20 = -sy
    r21 = cy * sx
    r22 = cy * cx

    px, py, pz = p(3), p(4), p(5)

    cax = Z[:, 0, :]
    cay = Z[:, 1, :] - f32(_GY)
    caz = Z[:, 2, :]

    contact = [(p(6 + i) < f32(0.1)).astype(f32) for i in range(2)]
    s = contact[0] + contact[1]
    active = (s > f32(0.0)).astype(f32)
    inv_s = jnp.where(s > f32(0.0), f32(1.0) / jnp.maximum(s, f32(1.0)), f32(0.0))
    fax, fay, faz = cax * inv_s, cay * inv_s, caz * inv_s

    planes = []
    f_planes, c_planes = [], []
    for i in range(2):
        ci = contact[i]
        fx, fy, fz = ci * fax, ci * fay, ci * faz

        f_planes += [r00 * fx + r10 * fy + r20 * fz,
                     r01 * fx + r11 * fy + r21 * fz,
                     r02 * fx + r12 * fy + r22 * fz]

        wcx, wcy, wcz = p(8 + 3 * i), p(9 + 3 * i), p(10 + 3 * i)
        dx, dy, dz = wcx - px, wcy - py, wcz - pz
        c_planes += [active * (r00 * dx + r10 * dy + r20 * dz),
                     active * (r01 * dx + r11 * dy + r21 * dz),
                     active * (r02 * dx + r12 * dy + r22 * dz)]

        mx = wcy * fz - wcz * fy
        my = wcz * fx - wcx * fz
        mz = wcx * fy - wcy * fx

        bfx = r00 * fx + r01 * fy + r02 * fz
        bfy = r10 * fx + r11 * fy + r12 * fz
        bfz = r20 * fx + r21 * fy + r22 * fz
        planes += [r00 * mx + r01 * my + r02 * mz + (py * bfz - pz * bfy),
                   r10 * mx + r11 * my + r12 * mz + (pz * bfx - px * bfz),
                   r20 * mx + r21 * my + r22 * mz + (px * bfy - py * bfx),
                   bfx, bfy, bfz]
    planes += f_planes + c_planes

    T24 = jnp.stack(planes)
    zpad = jnp.zeros((32 - 24, nr), f32)
    pieces = []
    for j in range(_PH):
        pieces.append(T24[:, j, :])
        pieces.append(zpad)
    out_ref[...] = jnp.concatenate(pieces, axis=0).T


@jax.jit
def _contact_call(pos, acc):
    B, T, D = pos.shape
    TB = 1024
    T_pad = -(-T // TB) * TB
    if T_pad != T:
        padw = ((0, 0), (0, T_pad - T), (0, 0))
        pos = jnp.pad(pos, padw)
        acc = jnp.pad(acc, padw)
    n_t = T_pad // TB
    nr = TB // _PH

    pos_v = pos.reshape(B, T_pad // _PH, _PH * D)
    acc_v = acc.reshape(B, T_pad // _PH, _PH * D)

    idx = lambda b, t: (b, t, 0)
    packed = pl.pallas_call(
        _fused_body,
        grid=(B, n_t),
        in_specs=[pl.BlockSpec((None, nr, _PH * D), idx),
                  pl.BlockSpec((None, nr, _PH * D), idx)],
        out_specs=pl.BlockSpec((None, nr, _PH * D), idx),
        out_shape=jax.ShapeDtypeStruct((B, T_pad // _PH, _PH * D), jnp.float32),
        compiler_params=pltpu.CompilerParams(
            dimension_semantics=("parallel", "parallel")),
    )(pos_v, acc_v)

    V = packed.reshape(B, T_pad, D)
    if T_pad != T:
        V = V[:, :T]
    return V[..., 0:12], V[..., 12:18], V[..., 18:24]


def kernel(pos, vel, acc):
    del vel
    B, T, D = pos.shape
    wrench, force, cop = _contact_call(pos.astype(jnp.float32),
                                       acc.astype(jnp.float32))
    zeros = lambda f: jnp.zeros((B, T, f), jnp.float32)
    return {
        "groundContactWrenchesInRootFrame": wrench,
        "groundContactForcesInRootFrame": force,
        "groundContactCenterOfPressureInRootFrame": cop,
        "groundContactTorquesInRootFrame": zeros(6),
        "residualWrenchInRootFrame": zeros(6),
        "contact": zeros(2),
        "comAccInRootFrame": zeros(3),
        "tau": zeros(D),
    }


# speedup vs baseline: 3.2412x; 3.2275x over previous
import functools

import jax
import jax.numpy as jnp
from jax.experimental import pallas as pl
from jax.experimental.pallas import tpu as pltpu

LANE = 128
SUB = 8
_GY = -9.81


def _fused_body(pos_ref, acc_ref, w_ref, f_ref, c_ref):
    f32 = jnp.float32
    X = pos_ref[...]
    A = acc_ref[...]

    def p(d):
        return X[d]

    ea, eb, ec = p(0), p(1), p(2)
    sx, cx = jnp.sin(ea), jnp.cos(ea)
    sy, cy = jnp.sin(eb), jnp.cos(eb)
    sz, cz = jnp.sin(ec), jnp.cos(ec)
    r00 = cz * cy
    r01 = cz * sy * sx - sz * cx
    r02 = cz * sy * cx + sz * sx
    r10 = sz * cy
    r11 = sz * sy * sx + cz * cx
    r12 = sz * sy * cx - cz * sx
    r20 = -sy
    r21 = cy * sx
    r22 = cy * cx

    px, py, pz = p(3), p(4), p(5)

    cax = A[0]
    cay = A[1] - f32(_GY)
    caz = A[2]

    contact = [(p(6 + i) < f32(0.1)).astype(f32) for i in range(2)]
    s = contact[0] + contact[1]
    active = (s > f32(0.0)).astype(f32)
    inv_s = jnp.where(s > f32(0.0), f32(1.0) / jnp.maximum(s, f32(1.0)), f32(0.0))
    fax, fay, faz = cax * inv_s, cay * inv_s, caz * inv_s

    w_planes, f_planes, c_planes = [], [], []
    for i in range(2):
        ci = contact[i]
        fx, fy, fz = ci * fax, ci * fay, ci * faz

        f_planes += [r00 * fx + r10 * fy + r20 * fz,
                     r01 * fx + r11 * fy + r21 * fz,
                     r02 * fx + r12 * fy + r22 * fz]

        wcx, wcy, wcz = p(8 + 3 * i), p(9 + 3 * i), p(10 + 3 * i)
        dx, dy, dz = wcx - px, wcy - py, wcz - pz
        c_planes += [active * (r00 * dx + r10 * dy + r20 * dz),
                     active * (r01 * dx + r11 * dy + r21 * dz),
                     active * (r02 * dx + r12 * dy + r22 * dz)]

        mx = wcy * fz - wcz * fy
        my = wcz * fx - wcx * fz
        mz = wcx * fy - wcy * fx

        bfx = r00 * fx + r01 * fy + r02 * fz
        bfy = r10 * fx + r11 * fy + r12 * fz
        bfz = r20 * fx + r21 * fy + r22 * fz
        w_planes += [r00 * mx + r01 * my + r02 * mz + (py * bfz - pz * bfy),
                     r10 * mx + r11 * my + r12 * mz + (pz * bfx - px * bfz),
                     r20 * mx + r21 * my + r22 * mz + (px * bfy - py * bfx),
                     bfx, bfy, bfz]

    w_ref[...] = jnp.stack(w_planes)
    f_ref[...] = jnp.stack(f_planes)
    c_ref[...] = jnp.stack(c_planes)


@jax.jit
def _contact_call(pos, acc):
    B, T, D = pos.shape
    TB = SUB * LANE
    T_pad = -(-T // TB) * TB
    if T_pad != T:
        padw = ((0, 0), (0, T_pad - T), (0, 0))
        pos = jnp.pad(pos, padw)
        acc = jnp.pad(acc, padw)
    n_t = T_pad // TB
    n_chunks = T_pad // LANE

    pos_p = jnp.swapaxes(pos, 1, 2)[:, 0:16, :].reshape(B, 16, n_chunks, LANE)
    acc_p = jnp.swapaxes(acc, 1, 2)[:, 0:8, :].reshape(B, 8, n_chunks, LANE)

    idx = lambda b, t: (b, 0, t, 0)
    wp, fp, cp = pl.pallas_call(
        _fused_body,
        grid=(B, n_t),
        in_specs=[pl.BlockSpec((None, 16, SUB, LANE), idx),
                  pl.BlockSpec((None, 8, SUB, LANE), idx)],
        out_specs=(pl.BlockSpec((None, 12, SUB, LANE), idx),
                   pl.BlockSpec((None, 6, SUB, LANE), idx),
                   pl.BlockSpec((None, 6, SUB, LANE), idx)),
        out_shape=(jax.ShapeDtypeStruct((B, 12, n_chunks, LANE), jnp.float32),
                   jax.ShapeDtypeStruct((B, 6, n_chunks, LANE), jnp.float32),
                   jax.ShapeDtypeStruct((B, 6, n_chunks, LANE), jnp.float32)),
        compiler_params=pltpu.CompilerParams(
            dimension_semantics=("parallel", "parallel")),
    )(pos_p, acc_p)

    def unpack(x, f):
        return jnp.swapaxes(x.reshape(B, f, T_pad), 1, 2)[:, :T]

    return unpack(wp, 12), unpack(fp, 6), unpack(cp, 6)


def kernel(pos, vel, acc):
    del vel
    B, T, D = pos.shape
    wrench, force, cop = _contact_call(pos.astype(jnp.float32),
                                       acc.astype(jnp.float32))
    zeros = lambda f: jnp.zeros((B, T, f), jnp.float32)
    return {
        "groundContactWrenchesInRootFrame": wrench,
        "groundContactForcesInRootFrame": force,
        "groundContactCenterOfPressureInRootFrame": cop,
        "groundContactTorquesInRootFrame": zeros(6),
        "residualWrenchInRootFrame": zeros(6),
        "contact": zeros(2),
        "comAccInRootFrame": zeros(3),
        "tau": zeros(D),
    }


# speedup vs baseline: 4.7996x; 1.4808x over previous
import functools

import jax
import jax.numpy as jnp
from jax.experimental import pallas as pl
from jax.experimental.pallas import tpu as pltpu

LANE = 128
SUB = 32
_GY = -9.81


def _fused_body(pos_ref, acc_ref, w_ref, f_ref, c_ref):
    f32 = jnp.float32
    X = pos_ref[...]
    A = acc_ref[...]

    def p(d):
        return X[d]

    ea, eb, ec = p(0), p(1), p(2)
    sx, cx = jnp.sin(ea), jnp.cos(ea)
    sy, cy = jnp.sin(eb), jnp.cos(eb)
    sz, cz = jnp.sin(ec), jnp.cos(ec)
    r00 = cz * cy
    r01 = cz * sy * sx - sz * cx
    r02 = cz * sy * cx + sz * sx
    r10 = sz * cy
    r11 = sz * sy * sx + cz * cx
    r12 = sz * sy * cx - cz * sx
    r20 = -sy
    r21 = cy * sx
    r22 = cy * cx

    px, py, pz = p(3), p(4), p(5)

    cax = A[0]
    cay = A[1] - f32(_GY)
    caz = A[2]

    contact = [(p(6 + i) < f32(0.1)).astype(f32) for i in range(2)]
    s = contact[0] + contact[1]
    active = (s > f32(0.0)).astype(f32)
    inv_s = jnp.where(s > f32(0.0), f32(1.0) / jnp.maximum(s, f32(1.0)), f32(0.0))
    fax, fay, faz = cax * inv_s, cay * inv_s, caz * inv_s

    w_planes, f_planes, c_planes = [], [], []
    for i in range(2):
        ci = contact[i]
        fx, fy, fz = ci * fax, ci * fay, ci * faz

        f_planes += [r00 * fx + r10 * fy + r20 * fz,
                     r01 * fx + r11 * fy + r21 * fz,
                     r02 * fx + r12 * fy + r22 * fz]

        wcx, wcy, wcz = p(8 + 3 * i), p(9 + 3 * i), p(10 + 3 * i)
        dx, dy, dz = wcx - px, wcy - py, wcz - pz
        c_planes += [active * (r00 * dx + r10 * dy + r20 * dz),
                     active * (r01 * dx + r11 * dy + r21 * dz),
                     active * (r02 * dx + r12 * dy + r22 * dz)]

        mx = wcy * fz - wcz * fy
        my = wcz * fx - wcx * fz
        mz = wcx * fy - wcy * fx

        bfx = r00 * fx + r01 * fy + r02 * fz
        bfy = r10 * fx + r11 * fy + r12 * fz
        bfz = r20 * fx + r21 * fy + r22 * fz
        w_planes += [r00 * mx + r01 * my + r02 * mz + (py * bfz - pz * bfy),
                     r10 * mx + r11 * my + r12 * mz + (pz * bfx - px * bfz),
                     r20 * mx + r21 * my + r22 * mz + (px * bfy - py * bfx),
                     bfx, bfy, bfz]

    w_ref[...] = jnp.stack(w_planes)
    f_ref[...] = jnp.stack(f_planes)
    c_ref[...] = jnp.stack(c_planes)


@jax.jit
def _contact_call(pos, acc):
    B, T, D = pos.shape
    TB = SUB * LANE
    T_pad = -(-T // TB) * TB
    if T_pad != T:
        padw = ((0, 0), (0, T_pad - T), (0, 0))
        pos = jnp.pad(pos, padw)
        acc = jnp.pad(acc, padw)
    n_t = T_pad // TB
    n_chunks = T_pad // LANE

    pos_p = jnp.swapaxes(pos, 1, 2)[:, 0:16, :].reshape(B, 16, n_chunks, LANE)
    acc_p = jnp.swapaxes(acc, 1, 2)[:, 0:8, :].reshape(B, 8, n_chunks, LANE)

    idx = lambda b, t: (b, 0, t, 0)
    wp, fp, cp = pl.pallas_call(
        _fused_body,
        grid=(B, n_t),
        in_specs=[pl.BlockSpec((None, 16, SUB, LANE), idx),
                  pl.BlockSpec((None, 8, SUB, LANE), idx)],
        out_specs=(pl.BlockSpec((None, 12, SUB, LANE), idx),
                   pl.BlockSpec((None, 6, SUB, LANE), idx),
                   pl.BlockSpec((None, 6, SUB, LANE), idx)),
        out_shape=(jax.ShapeDtypeStruct((B, 12, n_chunks, LANE), jnp.float32),
                   jax.ShapeDtypeStruct((B, 6, n_chunks, LANE), jnp.float32),
                   jax.ShapeDtypeStruct((B, 6, n_chunks, LANE), jnp.float32)),
        compiler_params=pltpu.CompilerParams(
            dimension_semantics=("parallel", "parallel")),
    )(pos_p, acc_p)

    def unpack(x, f):
        return jnp.swapaxes(x.reshape(B, f, T_pad), 1, 2)[:, :T]

    return unpack(wp, 12), unpack(fp, 6), unpack(cp, 6)


def kernel(pos, vel, acc):
    del vel
    B, T, D = pos.shape
    wrench, force, cop = _contact_call(pos.astype(jnp.float32),
                                       acc.astype(jnp.float32))
    zeros = lambda f: jnp.zeros((B, T, f), jnp.float32)
    return {
        "groundContactWrenchesInRootFrame": wrench,
        "groundContactForcesInRootFrame": force,
        "groundContactCenterOfPressureInRootFrame": cop,
        "groundContactTorquesInRootFrame": zeros(6),
        "residualWrenchInRootFrame": zeros(6),
        "contact": zeros(2),
        "comAccInRootFrame": zeros(3),
        "tau": zeros(D),
    }


# speedup vs baseline: 6.0662x; 1.2639x over previous
import functools

import jax
import jax.numpy as jnp
from jax.experimental import pallas as pl
from jax.experimental.pallas import tpu as pltpu

LANE = 128
SUB = 32
_GY = -9.81


def _fused_body(pos_ref, acc_ref, w_ref, f_ref, c_ref):
    f32 = jnp.float32
    TBL = pos_ref.shape[1]
    n = TBL // LANE
    X = pos_ref[...].reshape(16, n, LANE)
    A = acc_ref[...].reshape(8, n, LANE)

    def p(d):
        return X[d]

    ea, eb, ec = p(0), p(1), p(2)
    sx, cx = jnp.sin(ea), jnp.cos(ea)
    sy, cy = jnp.sin(eb), jnp.cos(eb)
    sz, cz = jnp.sin(ec), jnp.cos(ec)
    r00 = cz * cy
    r01 = cz * sy * sx - sz * cx
    r02 = cz * sy * cx + sz * sx
    r10 = sz * cy
    r11 = sz * sy * sx + cz * cx
    r12 = sz * sy * cx - cz * sx
    r20 = -sy
    r21 = cy * sx
    r22 = cy * cx

    px, py, pz = p(3), p(4), p(5)

    cax = A[0]
    cay = A[1] - f32(_GY)
    caz = A[2]

    contact = [(p(6 + i) < f32(0.1)).astype(f32) for i in range(2)]
    s = contact[0] + contact[1]
    active = (s > f32(0.0)).astype(f32)
    inv_s = jnp.where(s > f32(0.0), f32(1.0) / jnp.maximum(s, f32(1.0)), f32(0.0))
    fax, fay, faz = cax * inv_s, cay * inv_s, caz * inv_s

    w_planes, f_planes, c_planes = [], [], []
    for i in range(2):
        ci = contact[i]
        fx, fy, fz = ci * fax, ci * fay, ci * faz

        f_planes += [r00 * fx + r10 * fy + r20 * fz,
                     r01 * fx + r11 * fy + r21 * fz,
                     r02 * fx + r12 * fy + r22 * fz]

        wcx, wcy, wcz = p(8 + 3 * i), p(9 + 3 * i), p(10 + 3 * i)
        dx, dy, dz = wcx - px, wcy - py, wcz - pz
        c_planes += [active * (r00 * dx + r10 * dy + r20 * dz),
                     active * (r01 * dx + r11 * dy + r21 * dz),
                     active * (r02 * dx + r12 * dy + r22 * dz)]

        mx = wcy * fz - wcz * fy
        my = wcz * fx - wcx * fz
        mz = wcx * fy - wcy * fx

        bfx = r00 * fx + r01 * fy + r02 * fz
        bfy = r10 * fx + r11 * fy + r12 * fz
        bfz = r20 * fx + r21 * fy + r22 * fz
        w_planes += [r00 * mx + r01 * my + r02 * mz + (py * bfz - pz * bfy),
                     r10 * mx + r11 * my + r12 * mz + (pz * bfx - px * bfz),
                     r20 * mx + r21 * my + r22 * mz + (px * bfy - py * bfx),
                     bfx, bfy, bfz]

    w_ref[...] = jnp.stack(w_planes).reshape(12, TBL)
    f_ref[...] = jnp.stack(f_planes).reshape(6, TBL)
    c_ref[...] = jnp.stack(c_planes).reshape(6, TBL)


@jax.jit
def _contact_call(pos, acc):
    B, T, D = pos.shape
    TB = SUB * LANE
    T_pad = -(-T // TB) * TB
    if T_pad != T:
        padw = ((0, 0), (0, T_pad - T), (0, 0))
        pos = jnp.pad(pos, padw)
        acc = jnp.pad(acc, padw)
    n_pb = T_pad // TB

    pos2 = jnp.swapaxes(pos, 1, 2).reshape(B * D, T_pad)
    acc2 = jnp.swapaxes(acc, 1, 2).reshape(B * D, T_pad)

    wp, fp, cp = pl.pallas_call(
        _fused_body,
        grid=(B * n_pb,),
        in_specs=[pl.BlockSpec((16, TB), lambda k: (2 * (k // n_pb), k % n_pb)),
                  pl.BlockSpec((8, TB), lambda k: (4 * (k // n_pb), k % n_pb))],
        out_specs=(pl.BlockSpec((12, TB), lambda k: (0, k)),
                   pl.BlockSpec((6, TB), lambda k: (0, k)),
                   pl.BlockSpec((6, TB), lambda k: (0, k))),
        out_shape=(jax.ShapeDtypeStruct((12, B * T_pad), jnp.float32),
                   jax.ShapeDtypeStruct((6, B * T_pad), jnp.float32),
                   jax.ShapeDtypeStruct((6, B * T_pad), jnp.float32)),
        compiler_params=pltpu.CompilerParams(
            dimension_semantics=("parallel",)),
    )(pos2, acc2)

    def unpack(x, f):
        return jnp.transpose(x.reshape(f, B, T_pad), (1, 2, 0))[:, :T]

    return unpack(wp, 12), unpack(fp, 6), unpack(cp, 6)


def kernel(pos, vel, acc):
    del vel
    B, T, D = pos.shape
    wrench, force, cop = _contact_call(pos.astype(jnp.float32),
                                       acc.astype(jnp.float32))
    zeros = lambda f: jnp.zeros((B, T, f), jnp.float32)
    return {
        "groundContactWrenchesInRootFrame": wrench,
        "groundContactForcesInRootFrame": force,
        "groundContactCenterOfPressureInRootFrame": cop,
        "groundContactTorquesInRootFrame": zeros(6),
        "residualWrenchInRootFrame": zeros(6),
        "contact": zeros(2),
        "comAccInRootFrame": zeros(3),
        "tau": zeros(D),
    }
